# Initial kernel scaffold; baseline (speedup 1.0000x reference)
import jax
import jax.numpy as jnp
from jax.experimental import pallas as pl


def kernel(x, pe):
    raise NotImplementedError("write your pallas kernel here")


# speedup vs baseline: 3.5315x; 3.5315x over previous
import jax
import jax.numpy as jnp
from jax.experimental import pallas as pl


def kernel(x, pe):
    B, L, E = x.shape
    BL = 256
    nblk = L // BL
    last_off = L - (pe.shape[0] - 8)

    def body(x_ref, pe_ref, pn_ref, o_ref):
        j = pl.program_id(0)
        lo = pe_ref[...]
        extra = jnp.where(
            j == nblk - 1, pn_ref[last_off:last_off + 1, :], pn_ref[0:1, :]
        )
        rows = jnp.concatenate([lo[1:], extra], axis=0)
        o_ref[...] = x_ref[...] + rows[None, :, :]

    return pl.pallas_call(
        body,
        grid=(nblk,),
        in_specs=[
            pl.BlockSpec((B, BL, E), lambda j: (0, j, 0)),
            pl.BlockSpec((BL, E), lambda j: (j, 0)),
            pl.BlockSpec((8, E), lambda j: ((j + 1) * BL // 8, 0)),
        ],
        out_specs=pl.BlockSpec((B, BL, E), lambda j: (0, j, 0)),
        out_shape=jax.ShapeDtypeStruct(x.shape, x.dtype),
    )(x, pe, pe)
